# Initial kernel scaffold; baseline (speedup 1.0000x reference)
import jax
import jax.numpy as jnp
from jax.experimental import pallas as pl


def kernel(edge_index, edge_values, embeddings, W, b):
    raise NotImplementedError("write your pallas kernel here")


# speedup vs baseline: 5.4345x; 5.4345x over previous
import jax
import jax.numpy as jnp
from jax import lax
from jax.experimental import pallas as pl
from jax.experimental.pallas import tpu as pltpu
from jax.experimental.pallas import tpu_sc as plsc

N_NODES = 10000
D_FEAT = 128
N_EDGES = 320000

NC = 2
NS = 16
NW = NC * NS
LANES = 16
DBLK = D_FEAT // LANES

EPW = N_EDGES // NW
K = 128
NFULL = EPW // K
REM = EPW - NFULL * K

WPT = 624
TAIL = N_NODES - NS * WPT
ZROWS = 128


_GATHER_DNUMS = lax.GatherDimensionNumbers(
    offset_dims=(), collapsed_slice_dims=(0,), start_index_map=(0,))


def _bcast_lane(vv, j):
    return lax.gather(vv, jnp.full((LANES, 1), j, jnp.int32), _GATHER_DNUMS,
                      (1,), mode=lax.GatherScatterMode.PROMISE_IN_BOUNDS)


def _scale_rows(rows_buf, vals_v, nblocks):
    def blk(b, carry):
        vv = vals_v[pl.ds(b * LANES, LANES)]
        for j in range(LANES):
            e = b * LANES + j
            bv = _bcast_lane(vv, j)
            for d in range(DBLK):
                sl = pl.ds(d * LANES, LANES)
                rows_buf[e, sl] = rows_buf[e, sl] * bv
        return carry

    lax.fori_loop(0, nblocks, blk, 0)


def _sc_body(rows_hbm, cols_hbm, vals_hbm, emb_hbm, part_hbm,
             acc, idx_v, dst_v, vals_v, rows_buf,
             idx_r, dst_r, vals_r, rows_r, zbuf, sem):
    cid = lax.axis_index("c")
    sid = lax.axis_index("s")

    def zrow(i, carry):
        for d in range(DBLK):
            zbuf[i, pl.ds(d * LANES, LANES)] = jnp.zeros((LANES,), jnp.float32)
        return carry

    lax.fori_loop(0, ZROWS, zrow, 0)
    rbase = sid * WPT
    off = 0
    while off < WPT:
        zn = min(ZROWS, WPT - off)
        pltpu.sync_copy(zbuf.at[pl.ds(0, zn)], acc.at[pl.ds(rbase + off, zn)])
        off += zn

    @pl.when(sid == NS - 1)
    def _zero_tail():
        pltpu.sync_copy(zbuf.at[pl.ds(0, TAIL)],
                        acc.at[pl.ds(NS * WPT, TAIL)])

    plsc.subcore_barrier()

    ebase = (cid * NS + sid) * EPW

    def chunk(i, carry):
        base = ebase + i * K
        pltpu.sync_copy(cols_hbm.at[pl.ds(base, K)], idx_v)
        pltpu.sync_copy(rows_hbm.at[pl.ds(base, K)], dst_v)
        pltpu.sync_copy(vals_hbm.at[pl.ds(base, K)], vals_v)
        pltpu.async_copy(emb_hbm.at[idx_v], rows_buf, sem).wait()
        _scale_rows(rows_buf, vals_v, K // LANES)
        pltpu.sync_copy(rows_buf, acc.at[dst_v], add=True)
        return carry

    lax.fori_loop(0, NFULL, chunk, 0)

    if REM:
        base = ebase + NFULL * K
        pltpu.sync_copy(cols_hbm.at[pl.ds(base, REM)], idx_r)
        pltpu.sync_copy(rows_hbm.at[pl.ds(base, REM)], dst_r)
        pltpu.sync_copy(vals_hbm.at[pl.ds(base, REM)], vals_r)
        pltpu.async_copy(emb_hbm.at[idx_r], rows_r, sem).wait()
        _scale_rows(rows_r, vals_r, REM // LANES)
        pltpu.sync_copy(rows_r, acc.at[dst_r], add=True)

    plsc.subcore_barrier()

    pltpu.sync_copy(acc.at[pl.ds(rbase, WPT)],
                    part_hbm.at[cid, pl.ds(rbase, WPT)])

    @pl.when(sid == NS - 1)
    def _write_tail():
        pltpu.sync_copy(acc.at[pl.ds(NS * WPT, TAIL)],
                        part_hbm.at[cid, pl.ds(NS * WPT, TAIL)])


def _sc_partials(rows, cols, vals, embeddings):
    mesh = plsc.VectorSubcoreMesh(core_axis_name="c", subcore_axis_name="s",
                                  num_cores=NC, num_subcores=NS)
    f = pl.kernel(
        _sc_body,
        out_type=jax.ShapeDtypeStruct((NC, N_NODES, D_FEAT), jnp.float32),
        mesh=mesh,
        scratch_types=[
            pltpu.VMEM_SHARED((N_NODES, D_FEAT), jnp.float32),
            pltpu.VMEM((K,), jnp.int32),
            pltpu.VMEM((K,), jnp.int32),
            pltpu.VMEM((K,), jnp.float32),
            pltpu.VMEM((K, D_FEAT), jnp.float32),
            pltpu.VMEM((REM,), jnp.int32),
            pltpu.VMEM((REM,), jnp.int32),
            pltpu.VMEM((REM,), jnp.float32),
            pltpu.VMEM((REM, D_FEAT), jnp.float32),
            pltpu.VMEM((ZROWS, D_FEAT), jnp.float32),
            pltpu.SemaphoreType.DMA,
        ],
    )
    return f(rows, cols, vals, embeddings)


TCB = 1000


def _tc_body(emb_ref, p0_ref, p1_ref, w_ref, b_ref, o_ref):
    x = emb_ref[...] + p0_ref[...] + p1_ref[...]
    h = lax.dot_general(x, w_ref[...], (((1,), (1,)), ((), ())),
                        preferred_element_type=jnp.float32)
    h = h + b_ref[...]
    o_ref[...] = jnp.where(h >= 0, h, 0.01 * h)


def _tc_finish(embeddings, p0, p1, W, b2):
    blk = lambda i: (i, 0)
    return pl.pallas_call(
        _tc_body,
        grid=(N_NODES // TCB,),
        in_specs=[
            pl.BlockSpec((TCB, D_FEAT), blk),
            pl.BlockSpec((TCB, D_FEAT), blk),
            pl.BlockSpec((TCB, D_FEAT), blk),
            pl.BlockSpec((D_FEAT, D_FEAT), lambda i: (0, 0)),
            pl.BlockSpec((1, D_FEAT), lambda i: (0, 0)),
        ],
        out_specs=pl.BlockSpec((TCB, D_FEAT), blk),
        out_shape=jax.ShapeDtypeStruct((N_NODES, D_FEAT), jnp.float32),
    )(embeddings, p0, p1, W, b2)


def kernel(edge_index, edge_values, embeddings, W, b):
    rows = edge_index[0].astype(jnp.int32)
    cols = edge_index[1].astype(jnp.int32)
    vals = edge_values.astype(jnp.float32)
    emb = embeddings.astype(jnp.float32)
    part = _sc_partials(rows, cols, vals, emb)
    return _tc_finish(emb, part[0], part[1], W.astype(jnp.float32),
                      b.astype(jnp.float32).reshape(1, D_FEAT))
